# baseline (speedup 1.0000x reference)
import functools
import jax
import jax.numpy as jnp
from jax import lax
from jax.experimental import pallas as pl
from jax.experimental.pallas import tpu as pltpu
from jax.experimental.pallas import tpu_sc as plsc

VOCAB = 32000
ROWS = 1024
NC, NS, L = 2, 16, 16
NW = NC * NS
RPW = ROWS // NW
GROUP = 1
BUFW = GROUP * VOCAB
NDMA = RPW // GROUP


def _sc_body(ids_hbm, out_hbm, ids_v, buf, pos_v, val_v, fsem, ssem):
    wid = lax.axis_index("s") * NC + lax.axis_index("c")
    base = wid * RPW
    pltpu.sync_copy(ids_hbm.at[pl.ds(base, RPW)], ids_v)

    fill = jnp.full((L,), -20.0, jnp.float32)

    def fill_body(i, _):
        buf[pl.ds(i * L, L)] = fill
        return 0

    lax.fori_loop(0, BUFW // L, fill_body, 0)

    for g in range(NDMA):
        pltpu.make_async_copy(
            buf,
            out_hbm.at[pl.ds((base + g * GROUP) * VOCAB, BUFW)],
            fsem,
        ).start()

    lane = lax.iota(jnp.int32, L)
    v20 = jnp.full((L,), 20.0, jnp.float32)
    for h in range(RPW // L):
        ids = ids_v[pl.ds(h * L, L)]
        hot = lax.rem(ids + 1, VOCAB)
        pos_v[pl.ds(h * L, L)] = (base + h * L + lane) * VOCAB + hot
        val_v[pl.ds(h * L, L)] = v20

    for g in range(NDMA):
        pltpu.make_async_copy(
            buf,
            out_hbm.at[pl.ds((base + g * GROUP) * VOCAB, BUFW)],
            fsem,
        ).wait()

    pltpu.make_async_copy(val_v, out_hbm.at[pos_v], ssem).start()
    pltpu.make_async_copy(val_v, out_hbm.at[pos_v], ssem).wait()


_sc_kernel = functools.partial(
    pl.kernel,
    mesh=plsc.VectorSubcoreMesh(core_axis_name="c", subcore_axis_name="s"),
    out_type=jax.ShapeDtypeStruct((ROWS * VOCAB,), jnp.float32),
    scratch_types=[
        pltpu.VMEM((RPW,), jnp.int32),
        pltpu.VMEM((BUFW,), jnp.float32),
        pltpu.VMEM((RPW,), jnp.int32),
        pltpu.VMEM((RPW,), jnp.float32),
        pltpu.SemaphoreType.DMA,
        pltpu.SemaphoreType.DMA,
    ],
    compiler_params=pltpu.CompilerParams(needs_layout_passes=False),
)(_sc_body)


def kernel(input_ids):
    B, S = input_ids.shape
    ids = input_ids.reshape(B * S).astype(jnp.int32)
    out = _sc_kernel(ids)
    return out.reshape(B, S, VOCAB)


# speedup vs baseline: 3.9544x; 3.9544x over previous
import jax
import jax.numpy as jnp
from jax.experimental import pallas as pl
from jax.experimental.pallas import tpu as pltpu

VOCAB = 32000
ROW_BLK = 32


def _onehot_kernel(ids_ref, out_ref):
    col = jax.lax.broadcasted_iota(jnp.int32, (ROW_BLK, VOCAB), 1)
    nid = jax.lax.rem(ids_ref[...] + 1, VOCAB)
    out_ref[...] = jnp.where(col == nid, 20.0, -20.0)


def kernel(input_ids):
    B, S = input_ids.shape
    rows = B * S
    ids = input_ids.reshape(rows, 1).astype(jnp.int32)
    out = pl.pallas_call(
        _onehot_kernel,
        grid=(rows // ROW_BLK,),
        in_specs=[pl.BlockSpec((ROW_BLK, 1), lambda i: (i, 0))],
        out_specs=pl.BlockSpec((ROW_BLK, VOCAB), lambda i: (i, 0)),
        out_shape=jax.ShapeDtypeStruct((rows, VOCAB), jnp.float32),
        compiler_params=pltpu.CompilerParams(
            dimension_semantics=("parallel",),
        ),
    )(ids)
    return out.reshape(B, S, VOCAB)
